# baseline (speedup 1.0000x reference)
import functools

import jax
import jax.numpy as jnp
from jax import lax
from jax.experimental import pallas as pl
from jax.experimental.pallas import tpu as pltpu
from jax.experimental.pallas import tpu_sc as plsc

NUM_CORES = 2
NUM_SUBCORES = 16
NUM_WORKERS = NUM_CORES * NUM_SUBCORES
LANES = 16

CHUNK = 512
IDX_MINOR = 128
IDX_ROWS = CHUNK // IDX_MINOR
ROWS_PER_GROUP = 8

MAX_NORM = 1.0


def _body(x_hbm, table_hbm, out_hbm, idx_v, rows_v, sem):
    d_model = table_hbm.shape[1]
    rows_2d = rows_v
    wid = lax.axis_index("s") * NUM_CORES + lax.axis_index("c")
    n_total = out_hbm.shape[0]
    n_per_w = n_total // NUM_WORKERS
    steps = n_per_w // CHUNK
    xrow0 = wid * (n_per_w // IDX_MINOR)
    row0 = wid * n_per_w

    def step_fn(s, carry):
        pltpu.sync_copy(x_hbm.at[pl.ds(xrow0 + s * IDX_ROWS, IDX_ROWS)], idx_v)
        descs = [
            pltpu.async_copy(
                table_hbm.at[idx_v.at[j]],
                rows_2d.at[pl.ds(j * IDX_MINOR, IDX_MINOR)],
                sem,
            )
            for j in range(IDX_ROWS)
        ]
        for d in descs:
            d.wait()

        def group_fn(g, carry2):
            for rr in range(ROWS_PER_GROUP):
                r = g * ROWS_PER_GROUP + rr
                vs = [
                    rows_2d[r, pl.ds(j * LANES, LANES)]
                    for j in range(d_model // LANES)
                ]
                s01 = jnp.abs(vs[0]) + jnp.abs(vs[1])
                s23 = jnp.abs(vs[2]) + jnp.abs(vs[3])
                norm = jnp.broadcast_to(jnp.sum(s01 + s23), (LANES,))
                scale = jnp.where(
                    norm > MAX_NORM, MAX_NORM / (norm + 1e-7), jnp.float32(1.0)
                )
                for j in range(d_model // LANES):
                    rows_2d[r, pl.ds(j * LANES, LANES)] = vs[j] * scale
            return carry2

        lax.fori_loop(0, CHUNK // ROWS_PER_GROUP, group_fn, 0)
        pltpu.sync_copy(rows_2d, out_hbm.at[pl.ds(row0 + s * CHUNK, CHUNK)])
        return carry

    lax.fori_loop(0, steps, step_fn, 0)


def kernel(x, table):
    batch, hist = x.shape
    vocab, d_model = table.shape
    n_total = batch * hist
    x2d = x.reshape(n_total // IDX_MINOR, IDX_MINOR).astype(jnp.int32)

    mesh = plsc.VectorSubcoreMesh(
        core_axis_name="c",
        subcore_axis_name="s",
        num_cores=NUM_CORES,
        num_subcores=NUM_SUBCORES,
    )
    run = functools.partial(
        pl.kernel,
        out_type=jax.ShapeDtypeStruct((n_total, d_model), jnp.float32),
        mesh=mesh,
        compiler_params=pltpu.CompilerParams(
            needs_layout_passes=False, use_tc_tiling_on_sc=False
        ),
        scratch_types=[
            pltpu.VMEM((IDX_ROWS, IDX_MINOR), jnp.int32),
            pltpu.VMEM((CHUNK, d_model), jnp.float32),
            pltpu.SemaphoreType.DMA,
        ],
    )(_body)
    out = run(x2d, table)
    return out.reshape(batch, hist, d_model)
